# Initial kernel scaffold; baseline (speedup 1.0000x reference)
import jax
import jax.numpy as jnp
from jax.experimental import pallas as pl


def kernel(x, batch, W0, b0, W1, b1, W2, b2, W3, b3):
    raise NotImplementedError("write your pallas kernel here")


# speedup vs baseline: 9.3334x; 9.3334x over previous
import functools

import jax
import jax.numpy as jnp
from jax import lax
from jax.experimental import pallas as pl
from jax.experimental.pallas import tpu as pltpu

NUM_GRAPHS = 512


def _split_bf16(a):
    hi = a.astype(jnp.bfloat16)
    lo = (a - hi.astype(jnp.float32)).astype(jnp.bfloat16)
    return hi, lo


def _dot_f32(a, b):
    a_hi, a_lo = _split_bf16(a)
    b_hi, b_lo = _split_bf16(b)
    f32 = jnp.float32
    return (jnp.dot(a_hi, b_hi, preferred_element_type=f32)
            + jnp.dot(a_hi, b_lo, preferred_element_type=f32)
            + jnp.dot(a_lo, b_hi, preferred_element_type=f32))


def _pool_mlp_kernel(batch_ref, x_ref, w0_ref, b0_ref, w1_ref, b1_ref,
                     w2_ref, b2_ref, w3_ref, b3_ref, out_ref, acc_ref,
                     *, nb, bn):
    i = pl.program_id(0)
    ids = batch_ref[0, 0, :]
    onehot = (ids[:, None] == lax.broadcasted_iota(jnp.int32, (bn, NUM_GRAPHS), 1)
              ).astype(jnp.bfloat16)
    xf = x_ref[...]
    x_hi = xf.astype(jnp.bfloat16)
    x_lo = (xf - x_hi.astype(jnp.float32)).astype(jnp.bfloat16)
    dims = (((0,), (0,)), ((), ()))
    partial = (lax.dot_general(onehot, x_hi, dims,
                               preferred_element_type=jnp.float32)
               + lax.dot_general(onehot, x_lo, dims,
                                 preferred_element_type=jnp.float32))

    @pl.when(i == 0)
    def _():
        acc_ref[...] = partial

    @pl.when(i > 0)
    def _():
        acc_ref[...] += partial

    @pl.when(i == nb - 1)
    def _():
        g = acc_ref[...]
        h = jnp.maximum(
            jnp.dot(g, w0_ref[...], preferred_element_type=jnp.float32)
            + b0_ref[...], 0.0)
        h = jnp.maximum(
            jnp.dot(h, w1_ref[...], preferred_element_type=jnp.float32)
            + b1_ref[...], 0.0)
        h = jnp.maximum(
            jnp.dot(h, w2_ref[...], preferred_element_type=jnp.float32)
            + b2_ref[...], 0.0)
        out_ref[...] = (jnp.dot(h, w3_ref[...],
                                preferred_element_type=jnp.float32)
                        + b3_ref[...])


def kernel(x, batch, W0, b0, W1, b1, W2, b2, W3, b3):
    n, d = x.shape
    bn = 2000
    if n % bn:
        bn = max(k for k in range(8, min(n, 2048) + 1, 8) if n % k == 0)
    nb = n // bn

    batch32 = batch.astype(jnp.int32).reshape(nb, 1, bn)
    w3p = jnp.zeros((W3.shape[0], 128), jnp.float32).at[:, :1].set(W3)
    b3p = jnp.zeros((1, 128), jnp.float32).at[:, :1].set(b3[None, :])

    full = lambda s: pl.BlockSpec(s, lambda i: (0,) * len(s))
    out = pl.pallas_call(
        functools.partial(_pool_mlp_kernel, nb=nb, bn=bn),
        grid=(nb,),
        in_specs=[
            pl.BlockSpec((1, 1, bn), lambda i: (i, 0, 0)),
            pl.BlockSpec((bn, d), lambda i: (i, 0)),
            full(W0.shape), full((1, b0.shape[0])),
            full(W1.shape), full((1, b1.shape[0])),
            full(W2.shape), full((1, b2.shape[0])),
            full(w3p.shape), full(b3p.shape),
        ],
        out_specs=pl.BlockSpec((NUM_GRAPHS, 128), lambda i: (0, 0)),
        out_shape=jax.ShapeDtypeStruct((NUM_GRAPHS, 128), jnp.float32),
        scratch_shapes=[pltpu.VMEM((NUM_GRAPHS, d), jnp.float32)],
    )(batch32, x, W0, b0[None, :], W1, b1[None, :], W2, b2[None, :], w3p, b3p)
    return out[:, :1]
